# baseline (speedup 1.0000x reference)
import functools

import numpy as np
import jax
import jax.numpy as jnp
from jax import lax
from jax.experimental import pallas as pl
from jax.experimental.pallas import tpu as pltpu

N0 = 4096
DIN = 512
D = 256
DOUT = 40
E = 65536 + N0
KS = [4096, 3277, 1967, 984]
NPAD = [4096, 3328, 2048, 1024]
NEG = 5


def _mm(x, w, b, rs_in=None, rs_out=None):
    R, Cin = x.shape
    Cout = w.shape[1]
    BR = 256
    ins = [x, w, b.reshape(1, Cout)]
    specs = [
        pl.BlockSpec((BR, Cin), lambda i: (i, 0)),
        pl.BlockSpec((Cin, Cout), lambda i: (0, 0)),
        pl.BlockSpec((1, Cout), lambda i: (0, 0)),
    ]
    hin, hout = rs_in is not None, rs_out is not None
    if hin:
        ins.append(rs_in.reshape(R, 1))
        specs.append(pl.BlockSpec((BR, 1), lambda i: (i, 0)))
    if hout:
        ins.append(rs_out.reshape(R, 1))
        specs.append(pl.BlockSpec((BR, 1), lambda i: (i, 0)))

    def body(*refs):
        xv = refs[0][...]
        k = 3
        if hin:
            xv = xv * refs[k][...]
            k += 1
        acc = jnp.dot(xv, refs[1][...], preferred_element_type=jnp.float32)
        acc = acc + refs[2][...]
        if hout:
            acc = acc * refs[k][...]
        refs[-1][...] = acc

    return pl.pallas_call(
        body, grid=(R // BR,), in_specs=specs,
        out_specs=pl.BlockSpec((BR, Cout), lambda i: (i, 0)),
        out_shape=jax.ShapeDtypeStruct((R, Cout), jnp.float32))(*ins)


def _gcn(ct, y, s, skip=None, use_relu=True):
    Rs, R = ct.shape
    Dc = y.shape[1]
    BR = 256
    ins = [ct, y]
    specs = [pl.BlockSpec((Rs, BR), lambda i: (0, i)),
             pl.BlockSpec((Rs, Dc), lambda i: (0, 0))]
    hsc = s is not None
    if hsc:
        ins.append(s.reshape(R, 1))
        specs.append(pl.BlockSpec((BR, 1), lambda i: (i, 0)))
    hs = skip is not None
    if hs:
        ins.append(skip)
        specs.append(pl.BlockSpec((BR, Dc), lambda i: (i, 0)))

    def body(*refs):
        acc = lax.dot_general(refs[0][...], refs[1][...],
                              (((0,), (0,)), ((), ())),
                              preferred_element_type=jnp.float32)
        k = 2
        neigh = acc
        if hsc:
            neigh = acc * refs[k][...]
            k += 1
        xo = jax.nn.relu(neigh) if use_relu else neigh
        if hs:
            xo = xo + refs[k][...]
        refs[-2][...] = neigh
        refs[-1][...] = xo

    return pl.pallas_call(
        body, grid=(R // BR,), in_specs=specs,
        out_specs=[pl.BlockSpec((BR, Dc), lambda i: (i, 0))] * 2,
        out_shape=[jax.ShapeDtypeStruct((R, Dc), jnp.float32)] * 2)(*ins)


def _sel_comb(q, b, s, skip=None, use_relu=True):
    R, N = q.shape
    Dc = b.shape[1]
    BR = 256
    ins = [q, b, s.reshape(R, 1)]
    specs = [pl.BlockSpec((BR, N), lambda i: (i, 0)),
             pl.BlockSpec((N, Dc), lambda i: (0, 0)),
             pl.BlockSpec((BR, 1), lambda i: (i, 0))]
    hs = skip is not None
    if hs:
        ins.append(skip)
        specs.append(pl.BlockSpec((BR, Dc), lambda i: (i, 0)))

    def body(*refs):
        acc = jnp.dot(refs[0][...], refs[1][...],
                      preferred_element_type=jnp.float32)
        neigh = acc * refs[2][...]
        xo = jax.nn.relu(neigh) if use_relu else neigh
        if hs:
            xo = xo + refs[3][...]
        refs[-2][...] = neigh
        refs[-1][...] = xo

    return pl.pallas_call(
        body, grid=(R // BR,), in_specs=specs,
        out_specs=[pl.BlockSpec((BR, Dc), lambda i: (i, 0))] * 2,
        out_shape=[jax.ShapeDtypeStruct((R, Dc), jnp.float32)] * 2)(*ins)


def _mi_sums(xm, h_ext, nvalid):
    R, Dc = xm.shape
    BR = 256
    Hp = h_ext.shape[0]

    def body(xm_ref, h_ref, o_ref):
        g = pl.program_id(0)

        @pl.when(g == 0)
        def _():
            o_ref[...] = jnp.zeros_like(o_ref)

        xv = xm_ref[...]
        row = g * BR + lax.broadcasted_iota(jnp.int32, (BR, 1), 0)
        msk = (row < nvalid).astype(jnp.float32)
        hall = h_ref[pl.ds(g * BR, BR + 8), :]
        vals = []
        for sh in range(6):
            hblk = hall[5 - sh:5 - sh + BR, :]
            dsum = jnp.sum(xv * hblk, axis=1, keepdims=True)
            z = dsum if sh == 0 else -dsum
            vals.append(jnp.sum(jax.nn.log_sigmoid(z) * msk))
        zero = jnp.zeros((), jnp.float32)
        vec = jnp.stack(vals + [zero, zero])
        o_ref[...] = o_ref[...] + vec[:, None]

    return pl.pallas_call(
        body, grid=(R // BR,),
        in_specs=[pl.BlockSpec((BR, Dc), lambda i: (i, 0)),
                  pl.BlockSpec((Hp, Dc), lambda i: (0, 0))],
        out_specs=pl.BlockSpec((8, 128), lambda i: (0, 0)),
        out_shape=jax.ShapeDtypeStruct((8, 128), jnp.float32))(xm, h_ext)


def _recon_sum(x, ct):
    R, Dc = x.shape
    BR = 256
    CC = 1024

    def body(xb_ref, xf_ref, ct_ref, o_ref):
        g = pl.program_id(0)

        @pl.when(g == 0)
        def _():
            o_ref[...] = jnp.zeros_like(o_ref)

        xb = xb_ref[...]
        tot = jnp.zeros((), jnp.float32)
        for c in range(R // CC):
            xf = xf_ref[pl.ds(c * CC, CC), :]
            s = lax.dot_general(xb, xf, (((1,), (1,)), ((), ())),
                                preferred_element_type=jnp.float32)
            p = jnp.clip(jax.nn.sigmoid(s), 1e-7, 1.0 - 1e-7)
            t = ct_ref[:, pl.ds(c * CC, CC)] > 0.0
            tot = tot + jnp.sum(jnp.where(t, jnp.log(p), jnp.log(1.0 - p)))
        lane0 = lax.broadcasted_iota(jnp.int32, (8, 128), 0) == 0
        o_ref[...] = o_ref[...] + jnp.where(lane0, tot, 0.0)

    return pl.pallas_call(
        body, grid=(R // BR,),
        in_specs=[pl.BlockSpec((BR, Dc), lambda i: (i, 0)),
                  pl.BlockSpec((R, Dc), lambda i: (0, 0)),
                  pl.BlockSpec((BR, R), lambda i: (i, 0))],
        out_specs=pl.BlockSpec((8, 128), lambda i: (0, 0)),
        out_shape=jax.ShapeDtypeStruct((8, 128), jnp.float32))(x, x, ct)


def kernel(node_x, edge_index, neg_nums, params):
    ei = edge_index.astype(jnp.int32)
    sl = jnp.arange(N0, dtype=jnp.int32)
    src0 = jnp.concatenate([ei[0], sl])
    dst0 = jnp.concatenate([ei[1], sl])
    cw = N0 // 8
    dcol = []
    for q in range(8):
        off = dst0 - q * cw
        dcol.append(jnp.where((off >= 0) & (off < cw), off, cw))
    ct0 = jnp.concatenate(
        [jnp.zeros((N0, cw), jnp.float32)
         .at[src0, dcol[q]].add(1.0, mode='drop') for q in range(8)], axis=1)
    deg = jnp.zeros((N0,), jnp.float32).at[dst0].add(1.0)
    a = deg ** -0.5

    eW, eB = params['enc_W'], params['enc_b']
    dW, dB = params['dec_W'], params['dec_b']
    pw, pM = params['pool_w'], params['pool_M']

    iota0 = jnp.arange(N0, dtype=jnp.int32)
    zb = jnp.zeros((D,), jnp.float32)
    svec = [a]
    perm0 = [iota0]
    qf = [None]
    qft = [None]
    down_x = []
    mi_out = []
    x_in, rs_in = node_x, None

    for l in range(4):
        n_l, npad = KS[l], NPAD[l]
        y = _mm(x_in, eW[l], eB[l], rs_in=rs_in, rs_out=svec[l])
        if l == 0:
            neigh, xr = _gcn(ct0, y, svec[0])
        else:
            y_full = _mm(qft[l], y, zb)
            b_full, _ = _gcn(ct0, y_full, None, use_relu=False)
            neigh, xr = _sel_comb(qf[l], b_full, svec[l])
        if l == 3:
            x_cur = xr
            break
        down_x.append(xr)

        wcat = jnp.concatenate(
            [pM[l], pw[l], jnp.zeros((D, 127), jnp.float32)], axis=1)
        xmsc = _mm(xr, wcat, jnp.zeros((D + 128,), jnp.float32))
        xm, score = xmsc[:, :D], xmsc[:n_l, D]

        h_ext = jnp.concatenate([neigh[n_l - 5:n_l], neigh[:n_l]], axis=0)
        h_ext = jnp.pad(h_ext, ((0, npad + 8 - (n_l + 5)), (0, 0)))
        sums = _mi_sums(xm, h_ext, n_l)
        mi = -(sums[0, 0] / n_l)
        for j in range(NEG):
            mi = mi - (sums[1 + j, 0] / n_l) / neg_nums
        mi_out.append(mi)

        k_n, npad_n = KS[l + 1], NPAD[l + 1]
        top_vals, perm = lax.top_k(score, k_n)
        gate = jax.nn.sigmoid(top_vals)
        perm_pad = jnp.pad(perm, (0, npad_n - k_n))
        valid = jnp.arange(npad_n, dtype=jnp.int32) < k_n
        p0 = perm0[l][perm_pad]
        perm0.append(p0)
        svec.append(jnp.where(valid, a[p0], 0.0))
        qf.append((p0[:, None] == iota0[None, :]).astype(jnp.float32))
        qft.append((iota0[:, None] == p0[None, :]).astype(jnp.float32))

        selt = (perm_pad[:, None]
                == jnp.arange(npad, dtype=jnp.int32)[None, :]
                ).astype(jnp.float32)
        x_in = _mm(selt, xr, zb)
        rs_in = jnp.pad(gate, (0, npad_n - k_n))

    for li in range(3):
        u = 2 - li
        y_small = _mm(x_cur, dW[li], dB[li], rs_out=svec[u + 1])
        y_full = _mm(qft[u + 1], y_small, zb)
        if u == 0:
            _, x_cur = _gcn(ct0, y_full, svec[0], skip=down_x[0])
        else:
            b_full, _ = _gcn(ct0, y_full, None, use_relu=False)
            _, x_cur = _sel_comb(qf[u], b_full, svec[u], skip=down_x[u])

    struct_loss = -_recon_sum(x_cur, ct0)[0, 0] / (N0 * N0)

    wf = jnp.pad(dW[3], ((0, 0), (0, 128 - DOUT)))
    bf = jnp.pad(dB[3], (0, 128 - DOUT))
    yf = _mm(x_cur, wf, bf, rs_out=svec[0])
    neigh_f, _ = _gcn(ct0, yf, svec[0], use_relu=False)
    x_out = neigh_f[:, :DOUT]

    return x_out, mi_out, struct_loss


# speedup vs baseline: 1.5189x; 1.5189x over previous
import functools

import numpy as np
import jax
import jax.numpy as jnp
from jax import lax
from jax.experimental import pallas as pl
from jax.experimental.pallas import tpu as pltpu

N0 = 4096
DIN = 512
D = 256
DOUT = 40
E = 65536 + N0
KS = [4096, 3277, 1967, 984]
NPAD = [4096, 3328, 2048, 1024]
NEG = 5


def _mm(x, w, b, rs_in=None, rs_out=None):
    R, Cin = x.shape
    Cout = w.shape[1]
    BR = 256
    ins = [x, w, b.reshape(1, Cout)]
    specs = [
        pl.BlockSpec((BR, Cin), lambda i: (i, 0)),
        pl.BlockSpec((Cin, Cout), lambda i: (0, 0)),
        pl.BlockSpec((1, Cout), lambda i: (0, 0)),
    ]
    hin, hout = rs_in is not None, rs_out is not None
    if hin:
        ins.append(rs_in.reshape(R, 1))
        specs.append(pl.BlockSpec((BR, 1), lambda i: (i, 0)))
    if hout:
        ins.append(rs_out.reshape(R, 1))
        specs.append(pl.BlockSpec((BR, 1), lambda i: (i, 0)))

    def body(*refs):
        xv = refs[0][...]
        k = 3
        if hin:
            xv = xv * refs[k][...]
            k += 1
        acc = jnp.dot(xv, refs[1][...], preferred_element_type=jnp.float32)
        acc = acc + refs[2][...]
        if hout:
            acc = acc * refs[k][...]
        refs[-1][...] = acc

    return pl.pallas_call(
        body, grid=(R // BR,), in_specs=specs,
        out_specs=pl.BlockSpec((BR, Cout), lambda i: (i, 0)),
        out_shape=jax.ShapeDtypeStruct((R, Cout), jnp.float32))(*ins)


def _gcn(ct, y, s, skip=None, use_relu=True):
    Rs, R = ct.shape
    Dc = y.shape[1]
    BR = 256
    ins = [ct, y]
    specs = [pl.BlockSpec((Rs, BR), lambda i: (0, i)),
             pl.BlockSpec((Rs, Dc), lambda i: (0, 0))]
    hsc = s is not None
    if hsc:
        ins.append(s.reshape(R, 1))
        specs.append(pl.BlockSpec((BR, 1), lambda i: (i, 0)))
    hs = skip is not None
    if hs:
        ins.append(skip)
        specs.append(pl.BlockSpec((BR, Dc), lambda i: (i, 0)))

    def body(*refs):
        acc = lax.dot_general(refs[0][...], refs[1][...],
                              (((0,), (0,)), ((), ())),
                              preferred_element_type=jnp.float32)
        k = 2
        neigh = acc
        if hsc:
            neigh = acc * refs[k][...]
            k += 1
        xo = jax.nn.relu(neigh) if use_relu else neigh
        if hs:
            xo = xo + refs[k][...]
        refs[-2][...] = neigh
        refs[-1][...] = xo

    return pl.pallas_call(
        body, grid=(R // BR,), in_specs=specs,
        out_specs=[pl.BlockSpec((BR, Dc), lambda i: (i, 0))] * 2,
        out_shape=[jax.ShapeDtypeStruct((R, Dc), jnp.float32)] * 2)(*ins)


def _sel_comb(q, b, s, skip=None, use_relu=True):
    R, N = q.shape
    Dc = b.shape[1]
    BR = 256
    ins = [q, b, s.reshape(R, 1)]
    specs = [pl.BlockSpec((BR, N), lambda i: (i, 0)),
             pl.BlockSpec((N, Dc), lambda i: (0, 0)),
             pl.BlockSpec((BR, 1), lambda i: (i, 0))]
    hs = skip is not None
    if hs:
        ins.append(skip)
        specs.append(pl.BlockSpec((BR, Dc), lambda i: (i, 0)))

    def body(*refs):
        acc = jnp.dot(refs[0][...], refs[1][...],
                      preferred_element_type=jnp.float32)
        neigh = acc * refs[2][...]
        xo = jax.nn.relu(neigh) if use_relu else neigh
        if hs:
            xo = xo + refs[3][...]
        refs[-2][...] = neigh
        refs[-1][...] = xo

    return pl.pallas_call(
        body, grid=(R // BR,), in_specs=specs,
        out_specs=[pl.BlockSpec((BR, Dc), lambda i: (i, 0))] * 2,
        out_shape=[jax.ShapeDtypeStruct((R, Dc), jnp.float32)] * 2)(*ins)


def _mi_sums(xm, h_ext, nvalid):
    R, Dc = xm.shape
    BR = 256
    Hp = h_ext.shape[0]

    def body(xm_ref, h_ref, o_ref):
        g = pl.program_id(0)

        @pl.when(g == 0)
        def _():
            o_ref[...] = jnp.zeros_like(o_ref)

        xv = xm_ref[...]
        row = g * BR + lax.broadcasted_iota(jnp.int32, (BR, 1), 0)
        msk = (row < nvalid).astype(jnp.float32)
        hall = h_ref[pl.ds(g * BR, BR + 8), :]
        vals = []
        for sh in range(6):
            hblk = hall[5 - sh:5 - sh + BR, :]
            dsum = jnp.sum(xv * hblk, axis=1, keepdims=True)
            z = dsum if sh == 0 else -dsum
            vals.append(jnp.sum(jax.nn.log_sigmoid(z) * msk))
        zero = jnp.zeros((), jnp.float32)
        vec = jnp.stack(vals + [zero, zero])
        o_ref[...] = o_ref[...] + vec[:, None]

    return pl.pallas_call(
        body, grid=(R // BR,),
        in_specs=[pl.BlockSpec((BR, Dc), lambda i: (i, 0)),
                  pl.BlockSpec((Hp, Dc), lambda i: (0, 0))],
        out_specs=pl.BlockSpec((8, 128), lambda i: (0, 0)),
        out_shape=jax.ShapeDtypeStruct((8, 128), jnp.float32))(xm, h_ext)


def _recon_sum(x, ct):
    R, Dc = x.shape
    BR = 256
    CC = 1024

    def body(xb_ref, xf_ref, ct_ref, o_ref):
        g = pl.program_id(0)

        @pl.when(g == 0)
        def _():
            o_ref[...] = jnp.zeros_like(o_ref)

        xb = xb_ref[...]
        tot = jnp.zeros((), jnp.float32)
        for c in range(R // CC):
            xf = xf_ref[pl.ds(c * CC, CC), :]
            s = lax.dot_general(xb, xf, (((1,), (1,)), ((), ())),
                                preferred_element_type=jnp.float32)
            p = jnp.clip(jax.nn.sigmoid(s), 1e-7, 1.0 - 1e-7)
            t = ct_ref[:, pl.ds(c * CC, CC)] > 0.0
            tot = tot + jnp.sum(jnp.where(t, jnp.log(p), jnp.log(1.0 - p)))
        lane0 = lax.broadcasted_iota(jnp.int32, (8, 128), 0) == 0
        o_ref[...] = o_ref[...] + jnp.where(lane0, tot, 0.0)

    return pl.pallas_call(
        body, grid=(R // BR,),
        in_specs=[pl.BlockSpec((BR, Dc), lambda i: (i, 0)),
                  pl.BlockSpec((R, Dc), lambda i: (0, 0)),
                  pl.BlockSpec((BR, R), lambda i: (i, 0))],
        out_specs=pl.BlockSpec((8, 128), lambda i: (0, 0)),
        out_shape=jax.ShapeDtypeStruct((8, 128), jnp.float32))(x, x, ct)


def kernel(node_x, edge_index, neg_nums, params):
    ei = edge_index.astype(jnp.int32)
    sl = jnp.arange(N0, dtype=jnp.int32)
    src0 = jnp.concatenate([ei[0], sl])
    dst0 = jnp.concatenate([ei[1], sl])
    ct0 = jnp.zeros((N0, N0), jnp.float32).at[src0, dst0].add(1.0)
    deg = jnp.zeros((N0,), jnp.float32).at[dst0].add(1.0)
    a = deg ** -0.5

    eW, eB = params['enc_W'], params['enc_b']
    dW, dB = params['dec_W'], params['dec_b']
    pw, pM = params['pool_w'], params['pool_M']

    iota0 = jnp.arange(N0, dtype=jnp.int32)
    zb = jnp.zeros((D,), jnp.float32)
    svec = [a]
    perm0 = [iota0]
    qf = [None]
    qft = [None]
    down_x = []
    mi_out = []
    x_in, rs_in = node_x, None

    for l in range(4):
        n_l, npad = KS[l], NPAD[l]
        y = _mm(x_in, eW[l], eB[l], rs_in=rs_in, rs_out=svec[l])
        if l == 0:
            neigh, xr = _gcn(ct0, y, svec[0])
        else:
            y_full = _mm(qft[l], y, zb)
            b_full, _ = _gcn(ct0, y_full, None, use_relu=False)
            neigh, xr = _sel_comb(qf[l], b_full, svec[l])
        if l == 3:
            x_cur = xr
            break
        down_x.append(xr)

        wcat = jnp.concatenate(
            [pM[l], pw[l], jnp.zeros((D, 127), jnp.float32)], axis=1)
        xmsc = _mm(xr, wcat, jnp.zeros((D + 128,), jnp.float32))
        xm, score = xmsc[:, :D], xmsc[:n_l, D]

        h_ext = jnp.concatenate([neigh[n_l - 5:n_l], neigh[:n_l]], axis=0)
        h_ext = jnp.pad(h_ext, ((0, npad + 8 - (n_l + 5)), (0, 0)))
        sums = _mi_sums(xm, h_ext, n_l)
        mi = -(sums[0, 0] / n_l)
        for j in range(NEG):
            mi = mi - (sums[1 + j, 0] / n_l) / neg_nums
        mi_out.append(mi)

        k_n, npad_n = KS[l + 1], NPAD[l + 1]
        top_vals, perm = lax.top_k(score, k_n)
        gate = jax.nn.sigmoid(top_vals)
        perm_pad = jnp.pad(perm, (0, npad_n - k_n))
        valid = jnp.arange(npad_n, dtype=jnp.int32) < k_n
        p0 = perm0[l][perm_pad]
        perm0.append(p0)
        svec.append(jnp.where(valid, a[p0], 0.0))
        qf.append((p0[:, None] == iota0[None, :]).astype(jnp.float32))
        qft.append((iota0[:, None] == p0[None, :]).astype(jnp.float32))

        selt = (perm_pad[:, None]
                == jnp.arange(npad, dtype=jnp.int32)[None, :]
                ).astype(jnp.float32)
        x_in = _mm(selt, xr, zb)
        rs_in = jnp.pad(gate, (0, npad_n - k_n))

    for li in range(3):
        u = 2 - li
        y_small = _mm(x_cur, dW[li], dB[li], rs_out=svec[u + 1])
        y_full = _mm(qft[u + 1], y_small, zb)
        if u == 0:
            _, x_cur = _gcn(ct0, y_full, svec[0], skip=down_x[0])
        else:
            b_full, _ = _gcn(ct0, y_full, None, use_relu=False)
            _, x_cur = _sel_comb(qf[u], b_full, svec[u], skip=down_x[u])

    struct_loss = -_recon_sum(x_cur, ct0)[0, 0] / (N0 * N0)

    wf = jnp.pad(dW[3], ((0, 0), (0, 128 - DOUT)))
    bf = jnp.pad(dB[3], (0, 128 - DOUT))
    yf = _mm(x_cur, wf, bf, rs_out=svec[0])
    neigh_f, _ = _gcn(ct0, yf, svec[0], use_relu=False)
    x_out = neigh_f[:, :DOUT]

    return x_out, mi_out, struct_loss


# speedup vs baseline: 1.6383x; 1.0786x over previous
import functools

import numpy as np
import jax
import jax.numpy as jnp
from jax import lax
from jax.experimental import pallas as pl
from jax.experimental.pallas import tpu as pltpu

N0 = 4096
DIN = 512
D = 256
DOUT = 40
E = 65536 + N0
KS = [4096, 3277, 1967, 984]
NPAD = [4096, 3328, 2048, 1024]
NEG = 5


def _mm(x, w, b, rs_in=None, rs_out=None):
    R, Cin = x.shape
    Cout = w.shape[1]
    BR = 256
    ins = [x, w, b.reshape(1, Cout)]
    specs = [
        pl.BlockSpec((BR, Cin), lambda i: (i, 0)),
        pl.BlockSpec((Cin, Cout), lambda i: (0, 0)),
        pl.BlockSpec((1, Cout), lambda i: (0, 0)),
    ]
    hin, hout = rs_in is not None, rs_out is not None
    if hin:
        ins.append(rs_in.reshape(R, 1))
        specs.append(pl.BlockSpec((BR, 1), lambda i: (i, 0)))
    if hout:
        ins.append(rs_out.reshape(R, 1))
        specs.append(pl.BlockSpec((BR, 1), lambda i: (i, 0)))

    def body(*refs):
        xv = refs[0][...]
        k = 3
        if hin:
            xv = xv * refs[k][...]
            k += 1
        acc = jnp.dot(xv, refs[1][...], preferred_element_type=jnp.float32)
        acc = acc + refs[2][...]
        if hout:
            acc = acc * refs[k][...]
        refs[-1][...] = acc

    return pl.pallas_call(
        body, grid=(R // BR,), in_specs=specs,
        out_specs=pl.BlockSpec((BR, Cout), lambda i: (i, 0)),
        out_shape=jax.ShapeDtypeStruct((R, Cout), jnp.float32))(*ins)


def _gcn(ct, y, s, skip=None, use_relu=True):
    Rs, R = ct.shape
    Dc = y.shape[1]
    BR = 256
    ins = [ct, y]
    specs = [pl.BlockSpec((Rs, BR), lambda i: (0, i)),
             pl.BlockSpec((Rs, Dc), lambda i: (0, 0))]
    hsc = s is not None
    if hsc:
        ins.append(s.reshape(R, 1))
        specs.append(pl.BlockSpec((BR, 1), lambda i: (i, 0)))
    hs = skip is not None
    if hs:
        ins.append(skip)
        specs.append(pl.BlockSpec((BR, Dc), lambda i: (i, 0)))

    def body(*refs):
        acc = lax.dot_general(refs[0][...], refs[1][...],
                              (((0,), (0,)), ((), ())),
                              preferred_element_type=jnp.float32)
        k = 2
        neigh = acc
        if hsc:
            neigh = acc * refs[k][...]
            k += 1
        xo = jax.nn.relu(neigh) if use_relu else neigh
        if hs:
            xo = xo + refs[k][...]
        refs[-2][...] = neigh
        refs[-1][...] = xo

    return pl.pallas_call(
        body, grid=(R // BR,), in_specs=specs,
        out_specs=[pl.BlockSpec((BR, Dc), lambda i: (i, 0))] * 2,
        out_shape=[jax.ShapeDtypeStruct((R, Dc), jnp.float32)] * 2)(*ins)


def _sel_comb(q, b, s, skip=None, use_relu=True):
    R, N = q.shape
    Dc = b.shape[1]
    BR = 256
    ins = [q, b, s.reshape(R, 1)]
    specs = [pl.BlockSpec((BR, N), lambda i: (i, 0)),
             pl.BlockSpec((N, Dc), lambda i: (0, 0)),
             pl.BlockSpec((BR, 1), lambda i: (i, 0))]
    hs = skip is not None
    if hs:
        ins.append(skip)
        specs.append(pl.BlockSpec((BR, Dc), lambda i: (i, 0)))

    def body(*refs):
        acc = jnp.dot(refs[0][...], refs[1][...],
                      preferred_element_type=jnp.float32)
        neigh = acc * refs[2][...]
        xo = jax.nn.relu(neigh) if use_relu else neigh
        if hs:
            xo = xo + refs[3][...]
        refs[-2][...] = neigh
        refs[-1][...] = xo

    return pl.pallas_call(
        body, grid=(R // BR,), in_specs=specs,
        out_specs=[pl.BlockSpec((BR, Dc), lambda i: (i, 0))] * 2,
        out_shape=[jax.ShapeDtypeStruct((R, Dc), jnp.float32)] * 2)(*ins)


def _mi_sums(xm, h_ext, nvalid):
    R, Dc = xm.shape
    BR = 256
    Hp = h_ext.shape[0]

    def body(xm_ref, h_ref, o_ref):
        g = pl.program_id(0)

        @pl.when(g == 0)
        def _():
            o_ref[...] = jnp.zeros_like(o_ref)

        xv = xm_ref[...]
        row = g * BR + lax.broadcasted_iota(jnp.int32, (BR, 1), 0)
        msk = (row < nvalid).astype(jnp.float32)
        hall = h_ref[pl.ds(g * BR, BR + 8), :]
        vals = []
        for sh in range(6):
            hblk = hall[5 - sh:5 - sh + BR, :]
            dsum = jnp.sum(xv * hblk, axis=1, keepdims=True)
            z = dsum if sh == 0 else -dsum
            vals.append(jnp.sum(jax.nn.log_sigmoid(z) * msk))
        zero = jnp.zeros((), jnp.float32)
        vec = jnp.stack(vals + [zero, zero])
        o_ref[...] = o_ref[...] + vec[:, None]

    return pl.pallas_call(
        body, grid=(R // BR,),
        in_specs=[pl.BlockSpec((BR, Dc), lambda i: (i, 0)),
                  pl.BlockSpec((Hp, Dc), lambda i: (0, 0))],
        out_specs=pl.BlockSpec((8, 128), lambda i: (0, 0)),
        out_shape=jax.ShapeDtypeStruct((8, 128), jnp.float32))(xm, h_ext)


def _recon_sum(x, ct):
    R, Dc = x.shape
    BR = 256
    CC = 1024

    def body(xb_ref, xf_ref, ct_ref, o_ref):
        g = pl.program_id(0)

        @pl.when(g == 0)
        def _():
            o_ref[...] = jnp.zeros_like(o_ref)

        xb = xb_ref[...]
        tot = jnp.zeros((), jnp.float32)
        for c in range(R // CC):
            xf = xf_ref[pl.ds(c * CC, CC), :]
            s = lax.dot_general(xb, xf, (((1,), (1,)), ((), ())),
                                preferred_element_type=jnp.float32)
            p = jnp.clip(jax.nn.sigmoid(s), 1e-7, 1.0 - 1e-7)
            t = ct_ref[:, pl.ds(c * CC, CC)] > 0.0
            tot = tot + jnp.sum(jnp.where(t, jnp.log(p), jnp.log(1.0 - p)))
        lane0 = lax.broadcasted_iota(jnp.int32, (8, 128), 0) == 0
        o_ref[...] = o_ref[...] + jnp.where(lane0, tot, 0.0)

    return pl.pallas_call(
        body, grid=(R // BR,),
        in_specs=[pl.BlockSpec((BR, Dc), lambda i: (i, 0)),
                  pl.BlockSpec((R, Dc), lambda i: (0, 0)),
                  pl.BlockSpec((BR, R), lambda i: (i, 0))],
        out_specs=pl.BlockSpec((8, 128), lambda i: (0, 0)),
        out_shape=jax.ShapeDtypeStruct((8, 128), jnp.float32))(x, x, ct)


def kernel(node_x, edge_index, neg_nums, params):
    ei = edge_index.astype(jnp.int32)
    sl = jnp.arange(N0, dtype=jnp.int32)
    src0 = jnp.concatenate([ei[0], sl])
    dst0 = jnp.concatenate([ei[1], sl])
    ct0 = jnp.zeros((N0, N0), jnp.float32).at[src0, dst0].add(1.0)
    deg = jnp.sum(ct0, axis=0)
    a = deg ** -0.5

    eW, eB = params['enc_W'], params['enc_b']
    dW, dB = params['dec_W'], params['dec_b']
    pw, pM = params['pool_w'], params['pool_M']

    iota0 = jnp.arange(N0, dtype=jnp.int32)
    zb = jnp.zeros((D,), jnp.float32)
    svec = [a]
    perm0 = [iota0]
    qf = [None]
    qft = [None]
    down_x = []
    mi_out = []
    x_in, rs_in = node_x, None

    for l in range(4):
        n_l, npad = KS[l], NPAD[l]
        y = _mm(x_in, eW[l], eB[l], rs_in=rs_in, rs_out=svec[l])
        if l == 0:
            neigh, xr = _gcn(ct0, y, svec[0])
        else:
            y_full = _mm(qft[l], y, zb)
            b_full, _ = _gcn(ct0, y_full, None, use_relu=False)
            neigh, xr = _sel_comb(qf[l], b_full, svec[l])
        if l == 3:
            x_cur = xr
            break
        down_x.append(xr)

        wcat = jnp.concatenate(
            [pM[l], pw[l], jnp.zeros((D, 127), jnp.float32)], axis=1)
        xmsc = _mm(xr, wcat, jnp.zeros((D + 128,), jnp.float32))
        xm, score = xmsc[:, :D], xmsc[:n_l, D]

        h_ext = jnp.concatenate([neigh[n_l - 5:n_l], neigh[:n_l]], axis=0)
        h_ext = jnp.pad(h_ext, ((0, npad + 8 - (n_l + 5)), (0, 0)))
        sums = _mi_sums(xm, h_ext, n_l)
        mi = -(sums[0, 0] / n_l)
        for j in range(NEG):
            mi = mi - (sums[1 + j, 0] / n_l) / neg_nums
        mi_out.append(mi)

        k_n, npad_n = KS[l + 1], NPAD[l + 1]
        top_vals, perm = lax.top_k(score, k_n)
        gate = jax.nn.sigmoid(top_vals)
        perm_pad = jnp.pad(perm, (0, npad_n - k_n))
        valid = jnp.arange(npad_n, dtype=jnp.int32) < k_n
        p0 = perm0[l][perm_pad]
        perm0.append(p0)
        svec.append(jnp.where(valid, a[p0], 0.0))
        qf.append((p0[:, None] == iota0[None, :]).astype(jnp.float32))
        qft.append((iota0[:, None] == p0[None, :]).astype(jnp.float32))

        selt = (perm_pad[:, None]
                == jnp.arange(npad, dtype=jnp.int32)[None, :]
                ).astype(jnp.float32)
        x_in = _mm(selt, xr, zb)
        rs_in = jnp.pad(gate, (0, npad_n - k_n))

    for li in range(3):
        u = 2 - li
        y_small = _mm(x_cur, dW[li], dB[li], rs_out=svec[u + 1])
        y_full = _mm(qft[u + 1], y_small, zb)
        if u == 0:
            _, x_cur = _gcn(ct0, y_full, svec[0], skip=down_x[0])
        else:
            b_full, _ = _gcn(ct0, y_full, None, use_relu=False)
            _, x_cur = _sel_comb(qf[u], b_full, svec[u], skip=down_x[u])

    struct_loss = -_recon_sum(x_cur, ct0)[0, 0] / (N0 * N0)

    wf = jnp.pad(dW[3], ((0, 0), (0, 128 - DOUT)))
    bf = jnp.pad(dB[3], (0, 128 - DOUT))
    yf = _mm(x_cur, wf, bf, rs_out=svec[0])
    neigh_f, _ = _gcn(ct0, yf, svec[0], use_relu=False)
    x_out = neigh_f[:, :DOUT]

    return x_out, mi_out, struct_loss
